# baseline (speedup 1.0000x reference)
import functools

import jax
import jax.numpy as jnp
from jax import lax
from jax.experimental import pallas as pl
from jax.experimental.pallas import tpu as pltpu
from jax.experimental.pallas import tpu_sc as plsc

PAD_ID = 0
IGNORE_ID = -100
LANES = 16
NBUF = 3


def _sc_embed(dec_flat, wte, wpe, batch):
    N = dec_flat.shape[0]
    D = wte.shape[1]
    T = wpe.shape[0]
    info = plsc.get_sparse_core_info()
    nw = info.num_cores * info.num_subcores
    per_w = N // nw
    tw = T // nw
    C = 32
    hpb = tw // C
    n_chunks = per_w // C
    mesh = plsc.VectorSubcoreMesh(core_axis_name="c", subcore_axis_name="s")

    @functools.partial(
        pl.kernel,
        mesh=mesh,
        out_type=(
            jax.ShapeDtypeStruct((N, D), jnp.float32),
            jax.ShapeDtypeStruct((N,), jnp.int32),
            jax.ShapeDtypeStruct((N,), jnp.int32),
        ),
        scratch_types=[
            pltpu.VMEM((per_w,), jnp.int32),
        ] + [pltpu.VMEM((C,), jnp.int32) for _ in range(n_chunks)] + [
            pltpu.VMEM((n_chunks, C), jnp.int32),
            pltpu.VMEM((NBUF, C, D), jnp.float32),
            pltpu.VMEM((tw, D), jnp.float32),
        ] + [pltpu.SemaphoreType.DMA for _ in range(2 * NBUF + 3)],
    )
    def k(dec_hbm, wte_hbm, wpe_hbm, tok_hbm, din_hbm, keep_hbm,
          dec_v, *rest):
        din_vs = rest[:n_chunks]
        keep_v, rows_v, wpe_v = rest[n_chunks:n_chunks + 3]
        sems = rest[n_chunks + 3:]
        gsems = sems[:NBUF]
        osems = sems[NBUF:2 * NBUF]
        sw, sd0, sd1 = sems[2 * NBUF:]
        wid = lax.axis_index("s") * info.num_cores + lax.axis_index("c")
        t0 = pl.multiple_of(wid * tw, tw)

        wcp = pltpu.async_copy(wpe_hbm.at[pl.ds(t0, tw)], wpe_v, sw)

        def hbm_row(g):
            return pl.multiple_of((g // hpb) * T + t0 + (g % hpb) * C, C)

        dec_cp = []
        for b in range(batch):
            seg = pl.multiple_of(b * T + t0, tw)
            dec_cp.append(pltpu.async_copy(dec_hbm.at[pl.ds(seg, tw)],
                                           dec_v.at[pl.ds(b * tw, tw)], sd0))
        for cp in dec_cp:
            cp.wait()
        ign = jnp.full((LANES,), IGNORE_ID, jnp.int32)
        pad = jnp.full((LANES,), PAD_ID, jnp.int32)
        one = jnp.full((LANES,), 1, jnp.int32)

        def mask_block(g):
            for i in range(C // LANES):
                sl = pl.ds(i * LANES, LANES)
                v = dec_v[pl.ds(g * C + i * LANES, LANES)]
                m = v != ign
                din_vs[g][sl] = jnp.where(m, v, pad)
                keep_v[g, sl] = jnp.where(m, one, pad)

        gather_cp = [None] * NBUF
        out_cp = [None] * NBUF

        def start_chunk(g):
            b = g % NBUF
            gather_cp[b] = pltpu.async_copy(
                wte_hbm.at[din_vs[g]], rows_v.at[b], gsems[b])

        def finish_chunk(g):
            b = g % NBUF
            woff = (g % hpb) * C
            gather_cp[b].wait()

            @plsc.parallel_loop(0, C, step=1, unroll=4)
            def add_row(r):
                for j in range(D // LANES):
                    sl = pl.ds(j * LANES, LANES)
                    plsc.addupdate(rows_v.at[b, r, sl], wpe_v[woff + r, sl])
            out_cp[b] = pltpu.async_copy(
                rows_v.at[b], tok_hbm.at[pl.ds(hbm_row(g), C)], osems[b])

        for g in range(NBUF - 1):
            mask_block(g)
            start_chunk(g)
        for g in range(NBUF - 1, n_chunks):
            mask_block(g)
        small_cp = []
        for g in range(n_chunks):
            small_cp.append(pltpu.async_copy(
                din_vs[g], din_hbm.at[pl.ds(hbm_row(g), C)], sd0))
            small_cp.append(pltpu.async_copy(
                keep_v.at[g], keep_hbm.at[pl.ds(hbm_row(g), C)], sd1))
        wcp.wait()
        for g in range(n_chunks):
            nxt = g + NBUF - 1
            if nxt < n_chunks:
                if nxt >= NBUF:
                    out_cp[nxt % NBUF].wait()
                start_chunk(nxt)
            finish_chunk(g)
        for g in range(n_chunks - NBUF, n_chunks):
            out_cp[g % NBUF].wait()
        for cp in small_cp:
            cp.wait()

    return k(dec_flat, wte, wpe)


def kernel(enc_hid, dec_or_lab, metadata, wte, wpe):
    B, T = dec_or_lab.shape
    D = wte.shape[1]
    dec_flat = dec_or_lab.reshape(B * T)
    tok, din, keep = _sc_embed(dec_flat, wte, wpe[:T], B)
    token_emb = tok.reshape(B, T, D)
    keep_b = keep.reshape(B, T).astype(bool)
    dec_in = din.reshape(B, T)
    enc_mask_2d = jnp.ones((B, T), dtype=bool)
    return (enc_hid, token_emb, enc_mask_2d, keep_b, metadata, dec_in, keep_b)


# speedup vs baseline: 1.0506x; 1.0506x over previous
import functools

import jax
import jax.numpy as jnp
from jax import lax
from jax.experimental import pallas as pl
from jax.experimental.pallas import tpu as pltpu
from jax.experimental.pallas import tpu_sc as plsc

PAD_ID = 0
IGNORE_ID = -100
LANES = 16
NBUF = 3


def _sc_embed(dec_flat, wte, wpe, batch):
    N = dec_flat.shape[0]
    D = wte.shape[1]
    T = wpe.shape[0]
    info = plsc.get_sparse_core_info()
    nw = info.num_cores * info.num_subcores
    per_w = N // nw
    tw = T // nw
    C = 32
    hpb = tw // C
    n_chunks = per_w // C
    mesh = plsc.VectorSubcoreMesh(core_axis_name="c", subcore_axis_name="s")

    @functools.partial(
        pl.kernel,
        mesh=mesh,
        out_type=(
            jax.ShapeDtypeStruct((N, D), jnp.float32),
            jax.ShapeDtypeStruct((N,), jnp.int32),
            jax.ShapeDtypeStruct((N,), jnp.int32),
        ),
        scratch_types=[
            pltpu.VMEM((per_w,), jnp.int32),
        ] + [pltpu.VMEM((C,), jnp.int32) for _ in range(n_chunks)] + [
            pltpu.VMEM((n_chunks, C), jnp.int32),
            pltpu.VMEM((NBUF, C, D), jnp.float32),
            pltpu.VMEM((tw, D), jnp.float32),
        ] + [pltpu.SemaphoreType.DMA for _ in range(2 * NBUF + 3)],
    )
    def k(dec_hbm, wte_hbm, wpe_hbm, tok_hbm, din_hbm, keep_hbm,
          dec_v, *rest):
        din_vs = rest[:n_chunks]
        keep_v, rows_v, wpe_v = rest[n_chunks:n_chunks + 3]
        sems = rest[n_chunks + 3:]
        gsems = sems[:NBUF]
        osems = sems[NBUF:2 * NBUF]
        sw, sd0, sd1 = sems[2 * NBUF:]
        wid = lax.axis_index("s") * info.num_cores + lax.axis_index("c")
        t0 = pl.multiple_of(wid * tw, tw)

        wcp = pltpu.async_copy(wpe_hbm.at[pl.ds(t0, tw)], wpe_v, sw)

        def hbm_row(g):
            return pl.multiple_of((g // hpb) * T + t0 + (g % hpb) * C, C)

        dec_cp = []
        for b in range(batch):
            seg = pl.multiple_of(b * T + t0, tw)
            dec_cp.append(pltpu.async_copy(dec_hbm.at[pl.ds(seg, tw)],
                                           dec_v.at[pl.ds(b * tw, tw)], sd0))
        for cp in dec_cp:
            cp.wait()
        ign = jnp.full((LANES,), IGNORE_ID, jnp.int32)
        pad = jnp.full((LANES,), PAD_ID, jnp.int32)
        one = jnp.full((LANES,), 1, jnp.int32)

        def mask_block(g):
            for i in range(C // LANES):
                sl = pl.ds(i * LANES, LANES)
                v = dec_v[pl.ds(g * C + i * LANES, LANES)]
                m = v != ign
                din_vs[g][sl] = jnp.where(m, v, pad)
                keep_v[g, sl] = jnp.where(m, one, pad)

        gather_cp = [None] * NBUF
        out_cp = [None] * NBUF

        def start_chunk(g):
            b = g % NBUF
            gather_cp[b] = pltpu.async_copy(
                wte_hbm.at[din_vs[g]], rows_v.at[b], gsems[b])

        def finish_chunk(g):
            b = g % NBUF
            woff = (g % hpb) * C
            gather_cp[b].wait()

            @plsc.parallel_loop(0, C, step=1, unroll=2)
            def add_row(r):
                for j in range(D // LANES):
                    sl = pl.ds(j * LANES, LANES)
                    plsc.addupdate(rows_v.at[b, r, sl], wpe_v[woff + r, sl])
            out_cp[b] = pltpu.async_copy(
                rows_v.at[b], tok_hbm.at[pl.ds(hbm_row(g), C)], osems[b])

        for g in range(NBUF - 1):
            mask_block(g)
            start_chunk(g)
        for g in range(NBUF - 1, n_chunks):
            mask_block(g)
        small_cp = []
        for g in range(n_chunks):
            small_cp.append(pltpu.async_copy(
                din_vs[g], din_hbm.at[pl.ds(hbm_row(g), C)], sd0))
            small_cp.append(pltpu.async_copy(
                keep_v.at[g], keep_hbm.at[pl.ds(hbm_row(g), C)], sd1))
        wcp.wait()
        for g in range(n_chunks):
            nxt = g + NBUF - 1
            if nxt < n_chunks:
                if nxt >= NBUF:
                    out_cp[nxt % NBUF].wait()
                start_chunk(nxt)
            finish_chunk(g)
        for g in range(n_chunks - NBUF, n_chunks):
            out_cp[g % NBUF].wait()
        for cp in small_cp:
            cp.wait()

    return k(dec_flat, wte, wpe)


def kernel(enc_hid, dec_or_lab, metadata, wte, wpe):
    B, T = dec_or_lab.shape
    D = wte.shape[1]
    dec_flat = dec_or_lab.reshape(B * T)
    tok, din, keep = _sc_embed(dec_flat, wte, wpe[:T], B)
    token_emb = tok.reshape(B, T, D)
    keep_b = keep.reshape(B, T).astype(bool)
    dec_in = din.reshape(B, T)
    enc_mask_2d = jnp.ones((B, T), dtype=bool)
    return (enc_hid, token_emb, enc_mask_2d, keep_b, metadata, dec_in, keep_b)


# speedup vs baseline: 1.0747x; 1.0229x over previous
import functools

import jax
import jax.numpy as jnp
from jax import lax
from jax.experimental import pallas as pl
from jax.experimental.pallas import tpu as pltpu
from jax.experimental.pallas import tpu_sc as plsc

PAD_ID = 0
IGNORE_ID = -100
LANES = 16
NBUF = 3


def _sc_embed(dec, wte, wpe):
    batch, T = dec.shape
    N = batch * T
    D = wte.shape[1]
    info = plsc.get_sparse_core_info()
    nw = info.num_cores * info.num_subcores
    per_w = N // nw
    tw = T // nw
    C = 32
    hpb = tw // C
    n_chunks = per_w // C
    mesh = plsc.VectorSubcoreMesh(core_axis_name="c", subcore_axis_name="s")

    @functools.partial(
        pl.kernel,
        mesh=mesh,
        out_type=(
            jax.ShapeDtypeStruct((batch, T, D), jnp.float32),
            jax.ShapeDtypeStruct((batch, T), jnp.int32),
            jax.ShapeDtypeStruct((batch, T), jnp.int32),
        ),
        scratch_types=[
            pltpu.VMEM((per_w,), jnp.int32),
        ] + [pltpu.VMEM((C,), jnp.int32) for _ in range(n_chunks)] + [
            pltpu.VMEM((n_chunks, C), jnp.int32),
            pltpu.VMEM((NBUF, C, D), jnp.float32),
            pltpu.VMEM((tw, D), jnp.float32),
        ] + [pltpu.SemaphoreType.DMA for _ in range(2 * NBUF + 3)],
    )
    def k(dec_hbm, wte_hbm, wpe_hbm, tok_hbm, din_hbm, keep_hbm,
          dec_v, *rest):
        din_vs = rest[:n_chunks]
        keep_v, rows_v, wpe_v = rest[n_chunks:n_chunks + 3]
        sems = rest[n_chunks + 3:]
        gsems = sems[:NBUF]
        osems = sems[NBUF:2 * NBUF]
        sw, sd0, sd1 = sems[2 * NBUF:]
        wid = lax.axis_index("s") * info.num_cores + lax.axis_index("c")
        t0 = pl.multiple_of(wid * tw, tw)

        wcp = pltpu.async_copy(wpe_hbm.at[pl.ds(t0, tw)], wpe_v, sw)

        def hbm_pos(g):
            return g // hpb, pl.multiple_of(t0 + (g % hpb) * C, C)

        dec_cp = []
        for b in range(batch):
            dec_cp.append(pltpu.async_copy(dec_hbm.at[b, pl.ds(t0, tw)],
                                           dec_v.at[pl.ds(b * tw, tw)], sd0))
        for cp in dec_cp:
            cp.wait()
        ign = jnp.full((LANES,), IGNORE_ID, jnp.int32)
        pad = jnp.full((LANES,), PAD_ID, jnp.int32)
        one = jnp.full((LANES,), 1, jnp.int32)

        def mask_block(g):
            for i in range(C // LANES):
                sl = pl.ds(i * LANES, LANES)
                v = dec_v[pl.ds(g * C + i * LANES, LANES)]
                m = v != ign
                din_vs[g][sl] = jnp.where(m, v, pad)
                keep_v[g, sl] = jnp.where(m, one, pad)

        gather_cp = [None] * NBUF
        out_cp = [None] * NBUF

        def start_chunk(g):
            b = g % NBUF
            gather_cp[b] = pltpu.async_copy(
                wte_hbm.at[din_vs[g]], rows_v.at[b], gsems[b])

        def finish_chunk(g):
            b = g % NBUF
            woff = (g % hpb) * C
            gather_cp[b].wait()

            @plsc.parallel_loop(0, C, step=1, unroll=2)
            def add_row(r):
                for j in range(D // LANES):
                    sl = pl.ds(j * LANES, LANES)
                    plsc.addupdate(rows_v.at[b, r, sl], wpe_v[woff + r, sl])
            gb, gt = hbm_pos(g)
            out_cp[b] = pltpu.async_copy(
                rows_v.at[b], tok_hbm.at[gb, pl.ds(gt, C)], osems[b])

        for g in range(NBUF - 1):
            mask_block(g)
            start_chunk(g)
        for g in range(NBUF - 1, n_chunks):
            mask_block(g)
        small_cp = []
        for g in range(n_chunks):
            gb, gt = hbm_pos(g)
            small_cp.append(pltpu.async_copy(
                din_vs[g], din_hbm.at[gb, pl.ds(gt, C)], sd0))
            small_cp.append(pltpu.async_copy(
                keep_v.at[g], keep_hbm.at[gb, pl.ds(gt, C)], sd1))
        wcp.wait()
        for g in range(n_chunks):
            nxt = g + NBUF - 1
            if nxt < n_chunks:
                if nxt >= NBUF:
                    out_cp[nxt % NBUF].wait()
                start_chunk(nxt)
            finish_chunk(g)
        for g in range(n_chunks - NBUF, n_chunks):
            out_cp[g % NBUF].wait()
        for cp in small_cp:
            cp.wait()

    return k(dec, wte, wpe)


def kernel(enc_hid, dec_or_lab, metadata, wte, wpe):
    B, T = dec_or_lab.shape
    token_emb, dec_in, keep = _sc_embed(dec_or_lab, wte, wpe[:T])
    keep_b = keep.astype(bool)
    enc_mask_2d = jnp.ones((B, T), dtype=bool)
    return (enc_hid, token_emb, enc_mask_2d, keep_b, metadata, dec_in, keep_b)
